# Initial kernel scaffold; baseline (speedup 1.0000x reference)
import jax
import jax.numpy as jnp
from jax.experimental import pallas as pl


def kernel(inputs, labels):
    raise NotImplementedError("write your pallas kernel here")


# speedup vs baseline: 18.2592x; 18.2592x over previous
import jax
import jax.numpy as jnp
from jax.experimental import pallas as pl

N = 4096
DIM = 128
KNN = 10
MARGIN1 = 1.3
MARGIN2 = 0.5
ROWS = 256
GRID = N // ROWS
BIG = 1e30


def _edb_kernel(xb_ref, lb_ref, xa_ref, la_ref, out_ref):
    i = pl.program_id(0)
    xb = xb_ref[...]
    xa = xa_ref[...]
    lb = lb_ref[...]
    la = la_ref[...]

    g = jax.lax.dot_general(xb, xa, (((1,), (1,)), ((), ())),
                            preferred_element_type=jnp.float32)
    sq_b = jnp.sum(xb * xb, axis=1, keepdims=True)
    sq_a = jnp.sum(xa * xa, axis=1)[None, :]
    dist = jnp.sqrt(jnp.maximum(sq_b + sq_a - 2.0 * g, 1e-12))
    mask = lb == la

    s_same = jnp.sum(jnp.where(mask, dist, 0.0), axis=1, keepdims=True)
    c_same = jnp.sum(mask.astype(jnp.float32), axis=1, keepdims=True)

    iota = jax.lax.broadcasted_iota(jnp.int32, (ROWS, N), 1)
    d = dist
    vals = []
    ms = []
    for t in range(KNN):
        v = jnp.min(d, axis=1, keepdims=True)
        eq = d == v
        idx_any = jnp.min(jnp.where(eq, iota, N), axis=1, keepdims=True)
        idx_same = jnp.min(jnp.where(eq & mask, iota, N), axis=1,
                           keepdims=True)
        m = idx_same == idx_any
        vals.append(v)
        ms.append(m)
        if t < KNN - 1:
            d = jnp.where(iota == idx_any, BIG, d)

    border = vals[KNN - 1]
    zero = jnp.zeros_like(border)
    an_sum, an_cnt = zero, zero
    ae_sum, ae_cnt = zero, zero
    same_topk_sum = zero
    for t in range(KNN):
        v = vals[t]
        mf = ms[t].astype(jnp.float32)
        nf = 1.0 - mf
        an_sum = an_sum + nf * jnp.maximum(border - v + MARGIN1, 0.0)
        an_cnt = an_cnt + nf
        ae_sum = ae_sum + mf * jnp.maximum(MARGIN2 - v, 0.0)
        ae_cnt = ae_cnt + mf
        same_topk_sum = same_topk_sum + mf * v

    ap_cnt = c_same - ae_cnt
    ap_sum = s_same - same_topk_sum - border * ap_cnt
    ap_row = jnp.where(ap_cnt > 0, ap_sum / jnp.maximum(ap_cnt, 1.0), 0.0)
    an_row = jnp.where(an_cnt > 0, an_sum / jnp.maximum(an_cnt, 1.0), 0.0)
    ae_row = jnp.where(ae_cnt > 0, ae_sum / jnp.maximum(ae_cnt, 1.0), 0.0)

    part = jnp.concatenate([ap_row, an_row, ae_row], axis=1)
    part = jnp.sum(part, axis=0, keepdims=True)

    @pl.when(i == 0)
    def _init():
        out_ref[...] = jnp.zeros_like(out_ref)

    out_ref[...] += part

    @pl.when(i == GRID - 1)
    def _final():
        out_ref[...] = out_ref[...] * (1.0 / N)


def kernel(inputs, labels):
    lab = labels.astype(jnp.int32)
    out = pl.pallas_call(
        _edb_kernel,
        grid=(GRID,),
        in_specs=[
            pl.BlockSpec((ROWS, DIM), lambda i: (i, 0)),
            pl.BlockSpec((ROWS, 1), lambda i: (i, 0)),
            pl.BlockSpec((N, DIM), lambda i: (0, 0)),
            pl.BlockSpec((1, N), lambda i: (0, 0)),
        ],
        out_specs=pl.BlockSpec((1, 3), lambda i: (0, 0)),
        out_shape=jax.ShapeDtypeStruct((1, 3), jnp.float32),
    )(inputs, lab.reshape(N, 1), inputs, lab.reshape(1, N))
    return (out[0, 0], out[0, 1], out[0, 2])


# speedup vs baseline: 44.0563x; 2.4128x over previous
import jax
import jax.numpy as jnp
from jax.experimental import pallas as pl

N = 4096
DIM = 128
KNN = 10
MARGIN1 = 1.3
MARGIN2 = 0.5
ROWS = 256
GRID = N // ROWS
BIG = 1e30


def _edb_kernel(xb_ref, lb_ref, xa_ref, la_ref, out_ref):
    i = pl.program_id(0)
    xb = xb_ref[...]
    xa = xa_ref[...]
    lb = lb_ref[...]
    la = la_ref[...]

    g = jax.lax.dot_general(xb, xa, (((1,), (1,)), ((), ())),
                            preferred_element_type=jnp.float32)
    sq_b = jnp.sum(xb * xb, axis=1, keepdims=True)
    sq_a = jnp.sum(xa * xa, axis=1)[None, :]
    dist = jnp.sqrt(jnp.maximum(sq_b + sq_a - 2.0 * g, 1e-12))
    mask = lb == la

    bits = jax.lax.bitcast_convert_type(dist, jnp.int32)
    dp = jax.lax.bitcast_convert_type(
        (bits & jnp.int32(-2)) | mask.astype(jnp.int32), jnp.float32)

    s_same = jnp.sum(jnp.where(mask, dp, 0.0), axis=1, keepdims=True)
    c_same = jnp.sum(mask.astype(jnp.float32), axis=1, keepdims=True)

    vals = []
    prev = jnp.full((ROWS, 1), -1.0, jnp.float32)
    for _ in range(KNN):
        v = jnp.min(jnp.where(dp > prev, dp, BIG), axis=1, keepdims=True)
        vals.append(v)
        prev = v

    border_b = jax.lax.bitcast_convert_type(vals[KNN - 1], jnp.int32)
    border = jax.lax.bitcast_convert_type(border_b & jnp.int32(-2),
                                          jnp.float32)
    zero = jnp.zeros_like(border)
    an_sum, an_cnt = zero, zero
    ae_sum, ae_cnt = zero, zero
    same_topk_sum = zero
    for t in range(KNN):
        vb = jax.lax.bitcast_convert_type(vals[t], jnp.int32)
        mf = (vb & 1).astype(jnp.float32)
        v = jax.lax.bitcast_convert_type(vb & jnp.int32(-2), jnp.float32)
        nf = 1.0 - mf
        an_sum = an_sum + nf * jnp.maximum(border - v + MARGIN1, 0.0)
        an_cnt = an_cnt + nf
        ae_sum = ae_sum + mf * jnp.maximum(MARGIN2 - v, 0.0)
        ae_cnt = ae_cnt + mf
        same_topk_sum = same_topk_sum + mf * v

    ap_cnt = c_same - ae_cnt
    ap_sum = s_same - same_topk_sum - border * ap_cnt
    ap_row = jnp.where(ap_cnt > 0, ap_sum / jnp.maximum(ap_cnt, 1.0), 0.0)
    an_row = jnp.where(an_cnt > 0, an_sum / jnp.maximum(an_cnt, 1.0), 0.0)
    ae_row = jnp.where(ae_cnt > 0, ae_sum / jnp.maximum(ae_cnt, 1.0), 0.0)

    part = jnp.concatenate([ap_row, an_row, ae_row], axis=1)
    part = jnp.sum(part, axis=0, keepdims=True)

    @pl.when(i == 0)
    def _init():
        out_ref[...] = jnp.zeros_like(out_ref)

    out_ref[...] += part

    @pl.when(i == GRID - 1)
    def _final():
        out_ref[...] = out_ref[...] * (1.0 / N)


def kernel(inputs, labels):
    lab = labels.astype(jnp.int32)
    out = pl.pallas_call(
        _edb_kernel,
        grid=(GRID,),
        in_specs=[
            pl.BlockSpec((ROWS, DIM), lambda i: (i, 0)),
            pl.BlockSpec((ROWS, 1), lambda i: (i, 0)),
            pl.BlockSpec((N, DIM), lambda i: (0, 0)),
            pl.BlockSpec((1, N), lambda i: (0, 0)),
        ],
        out_specs=pl.BlockSpec((1, 3), lambda i: (0, 0)),
        out_shape=jax.ShapeDtypeStruct((1, 3), jnp.float32),
    )(inputs, lab.reshape(N, 1), inputs, lab.reshape(1, N))
    return (out[0, 0], out[0, 1], out[0, 2])
